# Initial kernel scaffold; baseline (speedup 1.0000x reference)
import jax
import jax.numpy as jnp
from jax.experimental import pallas as pl


def kernel(flattened_logits, list_mask):
    raise NotImplementedError("write your pallas kernel here")


# speedup vs baseline: 53.6717x; 53.6717x over previous
import functools
import math

import jax
import jax.numpy as jnp
from jax import lax
from jax.experimental import pallas as pl
from jax.experimental.pallas import tpu as pltpu
from jax.experimental.pallas import tpu_sc as plsc

_LOGEPS = math.log(1e-10)


def _make_kernel(B, L):
    NW = 32
    rows_per_w = B // NW
    BR = 64
    n_blocks = rows_per_w // BR
    blk_elems = BR * L
    NV = (L + 15) // 16
    buf_len = blk_elems + 16

    mesh = plsc.VectorSubcoreMesh(core_axis_name="c", subcore_axis_name="s")

    @functools.partial(
        pl.kernel,
        mesh=mesh,
        out_type=jax.ShapeDtypeStruct((B * L,), jnp.float32),
        compiler_params=pltpu.CompilerParams(needs_layout_passes=False),
        scratch_types=[
            pltpu.VMEM((buf_len,), jnp.float32),
            pltpu.VMEM((buf_len,), jnp.int32),
            pltpu.VMEM((buf_len,), jnp.float32),
            pltpu.VMEM((16 * NV,), jnp.float32),
        ],
    )
    def _k(x_hbm, m_hbm, out_hbm, xbuf, mbuf, obuf, accum):
        cid = lax.axis_index("c")
        sid = lax.axis_index("s")
        wid = sid * 2 + cid
        lane = lax.iota(jnp.int32, 16)
        zero16 = jnp.zeros((16,), jnp.float32)

        def row_body(r, _):
            base = r * L
            ms = []
            nv = jnp.int32(0)
            for v in range(NV):
                mv = mbuf[pl.ds(base + v * 16, 16)]
                if (v + 1) * 16 > L:
                    mv = jnp.where(v * 16 + lane < L, mv, 0)
                ms.append(mv)
                nv = nv + jnp.sum(mv)
            nvc = jnp.maximum(nv, 1)

            for v in range(NV):
                accum[pl.ds(v * 16, 16)] = zero16

            m_chunks = (L + nvc - 1) // nvc
            j_count = (nvc + 15) // 16

            def chunk_body(k, _):
                koff = k * nvc

                def j_body(j, _):
                    off = koff + j * 16
                    xv = xbuf[pl.ds(base + off, 16)]
                    ok = ((j * 16 + lane) < nvc) & ((off + lane) < L)
                    xv = jnp.where(ok, xv, 0.0)
                    plsc.addupdate(accum.at[pl.ds(j * 16, 16)], xv)
                    return 0

                lax.fori_loop(0, j_count, j_body, 0)
                return 0

            lax.fori_loop(0, m_chunks, chunk_body, 0)

            q = L // nvc
            rem = L - q * nvc
            scale = jnp.float32(2.0 ** -30)
            inv_q = (jnp.int32(1 << 30) // q).astype(jnp.float32) * scale
            inv_qp1 = ((jnp.int32(1 << 30) // (q + 1)).astype(jnp.float32)
                       * scale)
            carry = jnp.int32(0)
            nv_is0 = nv == 0
            for v in range(NV):
                mv = ms[v]
                inc = plsc.cumsum(mv)
                rank = (carry + inc) - mv
                carry = carry + jnp.sum(mv)
                acc_g = plsc.load_gather(accum, [rank])
                inv = jnp.where(rank < rem, inv_qp1, inv_q)
                eff = (mv == 1) | (nv_is0 & (v * 16 + lane == 0))
                res = jnp.where(eff, acc_g * inv, jnp.float32(_LOGEPS))
                obuf[pl.ds(base + v * 16, 16)] = res
            return 0

        def blk_body(b, _):
            start = (wid * rows_per_w * L) + b * blk_elems
            pltpu.sync_copy(x_hbm.at[pl.ds(start, blk_elems)],
                            xbuf.at[pl.ds(0, blk_elems)])
            pltpu.sync_copy(m_hbm.at[pl.ds(start, blk_elems)],
                            mbuf.at[pl.ds(0, blk_elems)])
            lax.fori_loop(0, BR, row_body, 0)
            pltpu.sync_copy(obuf.at[pl.ds(0, blk_elems)],
                            out_hbm.at[pl.ds(start, blk_elems)])
            return 0

        lax.fori_loop(0, n_blocks, blk_body, 0)

    return _k


@jax.jit
def kernel(flattened_logits, list_mask):
    B, L = list_mask.shape
    mask_i32 = list_mask.astype(jnp.int32).reshape(-1)
    out_flat = _make_kernel(B, L)(flattened_logits, mask_i32)
    return out_flat.reshape(B, L)
